# baseline (speedup 1.0000x reference)
import functools

import jax
import jax.numpy as jnp
from jax.experimental import pallas as pl

EPS = 1e-12


def _ln_kernel(x_ref, pe_ref, g_ref, b_ref, o_ref):
    xb = x_ref[...] + pe_ref[...][None, :, :]
    u = jnp.mean(xb, axis=-1, keepdims=True)
    d = xb - u
    s = jnp.mean(d * d, axis=-1, keepdims=True)
    o_ref[...] = g_ref[...] * (d * jax.lax.rsqrt(s + EPS)) + b_ref[...]


@functools.partial(jax.jit, static_argnames=("s_blk",))
def _run(x, pos_embed, gamma, beta, s_blk=256):
    B, S, D = x.shape
    gamma2 = gamma.reshape(1, D)
    beta2 = beta.reshape(1, D)
    grid = (S // s_blk,)
    return pl.pallas_call(
        _ln_kernel,
        grid=grid,
        in_specs=[
            pl.BlockSpec((B, s_blk, D), lambda s: (0, s, 0)),
            pl.BlockSpec((s_blk, D), lambda s: (s, 0)),
            pl.BlockSpec((1, D), lambda s: (0, 0)),
            pl.BlockSpec((1, D), lambda s: (0, 0)),
        ],
        out_specs=pl.BlockSpec((B, s_blk, D), lambda s: (0, s, 0)),
        out_shape=jax.ShapeDtypeStruct((B, S, D), x.dtype),
    )(x, pos_embed, gamma2, beta2)


def kernel(x, pos_embed, gamma, beta):
    S = x.shape[1]
    return _run(x, pos_embed[:S], gamma, beta)
